# baseline (speedup 1.0000x reference)
import functools

import jax
import jax.numpy as jnp
from jax.experimental import pallas as pl
from jax.experimental.pallas import tpu as pltpu


_CHUNK = 128


def _mm_kernel(a_ref, b_ref, o_ref):
    o_ref[...] = jnp.dot(a_ref[...], b_ref[...],
                         preferred_element_type=jnp.float32)


def _mm(a, b, tile_m=1024):
    M, K = a.shape
    N = b.shape[1]
    Mp = ((M + tile_m - 1) // tile_m) * tile_m
    if Mp != M:
        a = jnp.pad(a, ((0, Mp - M), (0, 0)))
    out = pl.pallas_call(
        _mm_kernel,
        grid=(Mp // tile_m,),
        in_specs=[pl.BlockSpec((tile_m, K), lambda i: (i, 0)),
                  pl.BlockSpec((K, N), lambda i: (0, 0))],
        out_specs=pl.BlockSpec((tile_m, N), lambda i: (i, 0)),
        out_shape=jax.ShapeDtypeStruct((Mp, N), jnp.float32),
    )(a, b)
    return out[:M]


def _gru_scan_kernel(x_ref, seg_ref, wih_ref, whh_ref, bih_ref, bhh_ref,
                     o_ref, h_ref, gi_ref, *, reverse):
    @pl.when(pl.program_id(0) == 0)
    def _init():
        h_ref[...] = jnp.zeros_like(h_ref)

    C, B, IN = x_ref.shape
    H = h_ref.shape[1]
    gi = jnp.dot(x_ref[...].reshape(C * B, IN), wih_ref[...],
                 preferred_element_type=jnp.float32) + bih_ref[...]
    gi_ref[...] = gi.reshape(C, B, 3 * H)

    def body(k, h):
        t = (C - 1 - k) if reverse else k
        gh = jnp.dot(h, whh_ref[...],
                     preferred_element_type=jnp.float32) + bhh_ref[...]
        g = gi_ref[pl.ds(t, 1)][0]
        r = jax.nn.sigmoid(g[:, :H] + gh[:, :H])
        z = jax.nn.sigmoid(g[:, H:2 * H] + gh[:, H:2 * H])
        n = jnp.tanh(g[:, 2 * H:] + r * gh[:, 2 * H:])
        hn = (1.0 - z) * n + z * h
        o_ref[pl.ds(t, 1), :, :] = hn[None]
        st = seg_ref[pl.ds(t, 1), :][0]
        return st[:, None] * hn

    h_ref[...] = jax.lax.fori_loop(0, C, body, h_ref[...])


def _gru_scan(x_tbi, seg_tb, p, reverse=False):
    Wih, Whh, bih, bhh = p
    Tp, B, IN = x_tbi.shape
    H = Whh.shape[1]
    G = Tp // _CHUNK
    if reverse:
        tmap = lambda i: (G - 1 - i, 0, 0)
        smap = lambda i: (G - 1 - i, 0)
    else:
        tmap = lambda i: (i, 0, 0)
        smap = lambda i: (i, 0)
    return pl.pallas_call(
        functools.partial(_gru_scan_kernel, reverse=reverse),
        grid=(G,),
        in_specs=[
            pl.BlockSpec((_CHUNK, B, IN), tmap),
            pl.BlockSpec((_CHUNK, B), smap),
            pl.BlockSpec((IN, 3 * H), lambda i: (0, 0)),
            pl.BlockSpec((H, 3 * H), lambda i: (0, 0)),
            pl.BlockSpec((1, 3 * H), lambda i: (0, 0)),
            pl.BlockSpec((1, 3 * H), lambda i: (0, 0)),
        ],
        out_specs=pl.BlockSpec((_CHUNK, B, H), tmap),
        out_shape=jax.ShapeDtypeStruct((Tp, B, H), jnp.float32),
        scratch_shapes=[pltpu.VMEM((B, H), jnp.float32),
                        pltpu.VMEM((_CHUNK, B, 3 * H), jnp.float32)],
    )(x_tbi, seg_tb, Wih.T, Whh.T, bih[None], bhh[None])


def _bigru_layer(x_bti, valid_tb, pf, pb, ones_tb):
    x_tbi = jnp.swapaxes(x_bti, 0, 1)
    yf = _gru_scan(x_tbi, ones_tb, pf)
    yb = _gru_scan(x_tbi, valid_tb, pb, reverse=True)
    return jnp.concatenate([jnp.swapaxes(yf, 0, 1),
                            jnp.swapaxes(yb, 0, 1)], axis=-1)


def _vq_kernel(z_ref, cbt_ref, c2_ref, cb_ref, q_ref, idx_ref):
    s = jnp.dot(z_ref[...], cbt_ref[...],
                preferred_element_type=jnp.float32) * (-2.0) + c2_ref[...]
    idx = jnp.argmin(s, axis=1).astype(jnp.int32)
    oh = (jax.lax.broadcasted_iota(jnp.int32, s.shape, 1)
          == idx[:, None]).astype(jnp.float32)
    q_ref[...] = jnp.dot(oh, cb_ref[...], preferred_element_type=jnp.float32)
    idx_ref[...] = idx[:, None]


def _vq(z2d, codebook, tile_m=1024):
    M, D = z2d.shape
    N = codebook.shape[0]
    Mp = ((M + tile_m - 1) // tile_m) * tile_m
    if Mp != M:
        z2d = jnp.pad(z2d, ((0, Mp - M), (0, 0)))
    c2 = jnp.sum(codebook * codebook, axis=1)[None, :]
    q, idx = pl.pallas_call(
        _vq_kernel,
        grid=(Mp // tile_m,),
        in_specs=[pl.BlockSpec((tile_m, D), lambda i: (i, 0)),
                  pl.BlockSpec((D, N), lambda i: (0, 0)),
                  pl.BlockSpec((1, N), lambda i: (0, 0)),
                  pl.BlockSpec((N, D), lambda i: (0, 0))],
        out_specs=[pl.BlockSpec((tile_m, D), lambda i: (i, 0)),
                   pl.BlockSpec((tile_m, 1), lambda i: (i, 0))],
        out_shape=[jax.ShapeDtypeStruct((Mp, D), jnp.float32),
                   jax.ShapeDtypeStruct((Mp, 1), jnp.int32)],
    )(z2d, codebook.T, c2, codebook)
    return q[:M], idx[:M, 0]


def _att_kernel(x_ref, wht_ref, bh_ref, wot_ref, bo_ref, mask_ref, o_ref):
    x = x_ref[0]
    h = jnp.tanh(jnp.dot(x, wht_ref[...],
                         preferred_element_type=jnp.float32) + bh_ref[...])
    a = jnp.dot(h, wot_ref[...],
                preferred_element_type=jnp.float32) + bo_ref[...]
    a = jnp.where(mask_ref[...] > 0, a, -1e30)
    amax = jnp.max(a, axis=0, keepdims=True)
    e = jnp.exp(a - amax)
    alpha = e / jnp.sum(e, axis=0, keepdims=True)
    pooled = jnp.sum(alpha * x, axis=0)
    nrm = jnp.sqrt(jnp.sum(pooled * pooled))
    o_ref[0, 0] = pooled / jnp.maximum(nrm, 1e-12)


def _att_norm(x_btd, att_p, tmax):
    B, Tp, D = x_btd.shape
    Wh, bh, Wo, bo = att_p
    A = Wh.shape[0]
    mask = (jnp.arange(Tp) < tmax).astype(jnp.float32)[:, None]
    return pl.pallas_call(
        _att_kernel,
        grid=(B,),
        in_specs=[
            pl.BlockSpec((1, Tp, D), lambda i: (i, 0, 0)),
            pl.BlockSpec((D, A), lambda i: (0, 0)),
            pl.BlockSpec((1, A), lambda i: (0, 0)),
            pl.BlockSpec((A, D), lambda i: (0, 0)),
            pl.BlockSpec((1, D), lambda i: (0, 0)),
            pl.BlockSpec((Tp, 1), lambda i: (0, 0)),
        ],
        out_specs=pl.BlockSpec((1, 1, D), lambda i: (i, 0, 0)),
        out_shape=jax.ShapeDtypeStruct((B, 1, D), jnp.float32),
    )(x_btd, Wh.T, bh[None], Wo.T, bo[None], mask)[:, 0]


def kernel(input, conv_w, conv_b, rnn0, pack, codebook, rnn1, att_p, l):
    B, Cin, L = input.shape
    O, _, K = conv_w.shape
    stride = 2
    T = (L - K) // stride + 1
    Tp = ((T + _CHUNK - 1) // _CHUNK) * _CHUNK
    l1 = (l - 4) // 2

    win = jnp.stack([input[:, :, k:k + stride * T:stride] for k in range(K)],
                    axis=-1)
    win = jnp.transpose(win, (0, 2, 3, 1)).reshape(B * T, K * Cin)
    wmat = jnp.transpose(conv_w, (2, 1, 0)).reshape(K * Cin, O)
    x = (_mm(win, wmat) + conv_b[None]).reshape(B, T, O)
    x = jnp.pad(x, ((0, 0), (0, Tp - T), (0, 0)))

    t_p = jnp.arange(Tp)
    valid1_tb = (t_p[:, None] < l1[None, :]).astype(jnp.float32)
    mask1 = jnp.swapaxes(valid1_tb, 0, 1)[:, :, None]
    ones_tb = jnp.ones((Tp, B), jnp.float32)

    h = x
    for pf, pb in rnn0:
        h = _bigru_layer(h, valid1_tb, pf, pb, ones_tb) * mask1

    D = codebook.shape[1]
    q2d, idx_flat = _vq(h.reshape(B * Tp, D), codebook)
    zq = q2d.reshape(B, Tp, D)
    idx = idx_flat.reshape(B, Tp)[:, :T]

    roll = jnp.roll(idx, 1, axis=1).at[:, 0].set(-1)
    seg = jnp.roll((idx == roll).astype(jnp.float32), -1, axis=1)
    Tmax1 = jnp.max(l1)
    tt = jnp.arange(T)
    seg = jnp.where(tt[None, :] == Tmax1 - 1, 0.0, seg)
    seg_p = jnp.pad(seg, ((0, 0), (0, Tp - T)))

    hs = _gru_scan(jnp.swapaxes(zq, 0, 1), jnp.swapaxes(seg_p, 0, 1), pack)
    hs = jnp.swapaxes(hs, 0, 1)[:, :T]

    m = (seg == 0) & (tt[None, :] < l1[:, None])
    counts = m.sum(1).astype(jnp.int32)
    src = jnp.argsort(jnp.where(m, tt[None, :], T), axis=1)
    packed = jnp.take_along_axis(hs, src[:, :, None], axis=1)
    packed = packed * (tt[None, :] < counts[:, None])[:, :, None]
    packed = jnp.pad(packed, ((0, 0), (0, Tp - T), (0, 0)))

    valid2_tb = (t_p[:, None] < counts[None, :]).astype(jnp.float32)
    mask2 = jnp.swapaxes(valid2_tb, 0, 1)[:, :, None]
    h2 = packed
    for pf, pb in rnn1:
        h2 = _bigru_layer(h2, valid2_tb, pf, pb, ones_tb) * mask2

    return _att_norm(h2, att_p, jnp.max(counts))


# speedup vs baseline: 1.2242x; 1.2242x over previous
import functools

import jax
import jax.numpy as jnp
from jax.experimental import pallas as pl
from jax.experimental.pallas import tpu as pltpu


_CHUNK = 128


def _mm_kernel(a_ref, b_ref, o_ref):
    o_ref[...] = jnp.dot(a_ref[...], b_ref[...],
                         preferred_element_type=jnp.float32)


def _mm(a, b, tile_m=1024):
    M, K = a.shape
    N = b.shape[1]
    Mp = ((M + tile_m - 1) // tile_m) * tile_m
    if Mp != M:
        a = jnp.pad(a, ((0, Mp - M), (0, 0)))
    out = pl.pallas_call(
        _mm_kernel,
        grid=(Mp // tile_m,),
        in_specs=[pl.BlockSpec((tile_m, K), lambda i: (i, 0)),
                  pl.BlockSpec((K, N), lambda i: (0, 0))],
        out_specs=pl.BlockSpec((tile_m, N), lambda i: (i, 0)),
        out_shape=jax.ShapeDtypeStruct((Mp, N), jnp.float32),
    )(a, b)
    return out[:M]


def _gru_scan_kernel(x_ref, seg_ref, wih_ref, whh_ref, bih_ref, bhh_ref,
                     o_ref, h_ref, gi_ref, *, reverse):
    @pl.when(pl.program_id(0) == 0)
    def _init():
        h_ref[...] = jnp.zeros_like(h_ref)

    C, B, IN = x_ref.shape
    H = h_ref.shape[1]
    gi = jnp.dot(x_ref[...].reshape(C * B, IN), wih_ref[...],
                 preferred_element_type=jnp.float32) + bih_ref[...]
    gi_ref[...] = gi.reshape(C, B, 3 * H)

    def body(k, h):
        t = (C - 1 - k) if reverse else k
        gh = jnp.dot(h, whh_ref[...],
                     preferred_element_type=jnp.float32) + bhh_ref[...]
        g = gi_ref[pl.ds(t, 1)][0]
        r = jax.nn.sigmoid(g[:, :H] + gh[:, :H])
        z = jax.nn.sigmoid(g[:, H:2 * H] + gh[:, H:2 * H])
        n = jnp.tanh(g[:, 2 * H:] + r * gh[:, 2 * H:])
        hn = (1.0 - z) * n + z * h
        o_ref[pl.ds(t, 1), :, :] = hn[None]
        st = seg_ref[pl.ds(t, 1), :][0]
        return st[:, None] * hn

    h_ref[...] = jax.lax.fori_loop(0, C, body, h_ref[...])


def _gru_scan(x_tbi, seg_tb, p, reverse=False):
    Wih, Whh, bih, bhh = p
    Tp, B, IN = x_tbi.shape
    H = Whh.shape[1]
    G = Tp // _CHUNK
    if reverse:
        tmap = lambda i: (G - 1 - i, 0, 0)
        smap = lambda i: (G - 1 - i, 0)
    else:
        tmap = lambda i: (i, 0, 0)
        smap = lambda i: (i, 0)
    return pl.pallas_call(
        functools.partial(_gru_scan_kernel, reverse=reverse),
        grid=(G,),
        in_specs=[
            pl.BlockSpec((_CHUNK, B, IN), tmap),
            pl.BlockSpec((_CHUNK, B), smap),
            pl.BlockSpec((IN, 3 * H), lambda i: (0, 0)),
            pl.BlockSpec((H, 3 * H), lambda i: (0, 0)),
            pl.BlockSpec((1, 3 * H), lambda i: (0, 0)),
            pl.BlockSpec((1, 3 * H), lambda i: (0, 0)),
        ],
        out_specs=pl.BlockSpec((_CHUNK, B, H), tmap),
        out_shape=jax.ShapeDtypeStruct((Tp, B, H), jnp.float32),
        scratch_shapes=[pltpu.VMEM((B, H), jnp.float32),
                        pltpu.VMEM((_CHUNK, B, 3 * H), jnp.float32)],
    )(x_tbi, seg_tb, Wih.T, Whh.T, bih[None], bhh[None])


def _bigru_layer(x_bti, valid_tb, pf, pb, ones_tb):
    x_tbi = jnp.swapaxes(x_bti, 0, 1)
    yf = _gru_scan(x_tbi, ones_tb, pf)
    yb = _gru_scan(x_tbi, valid_tb, pb, reverse=True)
    return jnp.concatenate([jnp.swapaxes(yf, 0, 1),
                            jnp.swapaxes(yb, 0, 1)], axis=-1)


def _vq_kernel(z_ref, cbt_ref, c2_ref, cb_ref, q_ref, idx_ref):
    s = jnp.dot(z_ref[...], cbt_ref[...],
                preferred_element_type=jnp.float32) * (-2.0) + c2_ref[...]
    idx = jnp.argmin(s, axis=1).astype(jnp.int32)
    oh = (jax.lax.broadcasted_iota(jnp.int32, s.shape, 1)
          == idx[:, None]).astype(jnp.float32)
    q_ref[...] = jnp.dot(oh, cb_ref[...], preferred_element_type=jnp.float32)
    idx_ref[...] = idx[:, None]


def _vq(z2d, codebook, tile_m=1024):
    M, D = z2d.shape
    N = codebook.shape[0]
    Mp = ((M + tile_m - 1) // tile_m) * tile_m
    if Mp != M:
        z2d = jnp.pad(z2d, ((0, Mp - M), (0, 0)))
    c2 = jnp.sum(codebook * codebook, axis=1)[None, :]
    q, idx = pl.pallas_call(
        _vq_kernel,
        grid=(Mp // tile_m,),
        in_specs=[pl.BlockSpec((tile_m, D), lambda i: (i, 0)),
                  pl.BlockSpec((D, N), lambda i: (0, 0)),
                  pl.BlockSpec((1, N), lambda i: (0, 0)),
                  pl.BlockSpec((N, D), lambda i: (0, 0))],
        out_specs=[pl.BlockSpec((tile_m, D), lambda i: (i, 0)),
                   pl.BlockSpec((tile_m, 1), lambda i: (i, 0))],
        out_shape=[jax.ShapeDtypeStruct((Mp, D), jnp.float32),
                   jax.ShapeDtypeStruct((Mp, 1), jnp.int32)],
    )(z2d, codebook.T, c2, codebook)
    return q[:M], idx[:M, 0]


def _att_kernel(x_ref, wht_ref, bh_ref, wot_ref, bo_ref, mask_ref, o_ref):
    x = x_ref[0]
    h = jnp.tanh(jnp.dot(x, wht_ref[...],
                         preferred_element_type=jnp.float32) + bh_ref[...])
    a = jnp.dot(h, wot_ref[...],
                preferred_element_type=jnp.float32) + bo_ref[...]
    a = jnp.where(mask_ref[...] > 0, a, -1e30)
    amax = jnp.max(a, axis=0, keepdims=True)
    e = jnp.exp(a - amax)
    alpha = e / jnp.sum(e, axis=0, keepdims=True)
    pooled = jnp.sum(alpha * x, axis=0)
    nrm = jnp.sqrt(jnp.sum(pooled * pooled))
    o_ref[0, 0] = pooled / jnp.maximum(nrm, 1e-12)


def _att_norm(x_btd, att_p, tmax):
    B, Tp, D = x_btd.shape
    Wh, bh, Wo, bo = att_p
    A = Wh.shape[0]
    mask = (jnp.arange(Tp) < tmax).astype(jnp.float32)[:, None]
    return pl.pallas_call(
        _att_kernel,
        grid=(B,),
        in_specs=[
            pl.BlockSpec((1, Tp, D), lambda i: (i, 0, 0)),
            pl.BlockSpec((D, A), lambda i: (0, 0)),
            pl.BlockSpec((1, A), lambda i: (0, 0)),
            pl.BlockSpec((A, D), lambda i: (0, 0)),
            pl.BlockSpec((1, D), lambda i: (0, 0)),
            pl.BlockSpec((Tp, 1), lambda i: (0, 0)),
        ],
        out_specs=pl.BlockSpec((1, 1, D), lambda i: (i, 0, 0)),
        out_shape=jax.ShapeDtypeStruct((B, 1, D), jnp.float32),
    )(x_btd, Wh.T, bh[None], Wo.T, bo[None], mask)[:, 0]


def kernel(input, conv_w, conv_b, rnn0, pack, codebook, rnn1, att_p, l):
    B, Cin, L = input.shape
    O, _, K = conv_w.shape
    stride = 2
    T = (L - K) // stride + 1
    Tp = ((T + _CHUNK - 1) // _CHUNK) * _CHUNK
    l1 = (l - 4) // 2

    idx_t = stride * jnp.arange(T)[:, None] + jnp.arange(K)[None, :]
    win = input[:, :, idx_t]
    win = jnp.transpose(win, (0, 2, 3, 1)).reshape(B * T, K * Cin)
    wmat = jnp.transpose(conv_w, (2, 1, 0)).reshape(K * Cin, O)
    x = (_mm(win, wmat) + conv_b[None]).reshape(B, T, O)
    x = jnp.pad(x, ((0, 0), (0, Tp - T), (0, 0)))

    t_p = jnp.arange(Tp)
    valid1_tb = (t_p[:, None] < l1[None, :]).astype(jnp.float32)
    mask1 = jnp.swapaxes(valid1_tb, 0, 1)[:, :, None]
    ones_tb = jnp.ones((Tp, B), jnp.float32)

    h = x
    for pf, pb in rnn0:
        h = _bigru_layer(h, valid1_tb, pf, pb, ones_tb) * mask1

    D = codebook.shape[1]
    q2d, idx_flat = _vq(h.reshape(B * Tp, D), codebook)
    zq = q2d.reshape(B, Tp, D)
    idx = idx_flat.reshape(B, Tp)[:, :T]

    roll = jnp.roll(idx, 1, axis=1).at[:, 0].set(-1)
    seg = jnp.roll((idx == roll).astype(jnp.float32), -1, axis=1)
    Tmax1 = jnp.max(l1)
    tt = jnp.arange(T)
    seg = jnp.where(tt[None, :] == Tmax1 - 1, 0.0, seg)
    seg_p = jnp.pad(seg, ((0, 0), (0, Tp - T)))

    hs = _gru_scan(jnp.swapaxes(zq, 0, 1), jnp.swapaxes(seg_p, 0, 1), pack)
    hs = jnp.swapaxes(hs, 0, 1)[:, :T]

    m = (seg == 0) & (tt[None, :] < l1[:, None])
    counts = m.sum(1).astype(jnp.int32)
    src = jnp.argsort(jnp.where(m, tt[None, :], T), axis=1)
    packed = jnp.take_along_axis(hs, src[:, :, None], axis=1)
    packed = packed * (tt[None, :] < counts[:, None])[:, :, None]
    packed = jnp.pad(packed, ((0, 0), (0, Tp - T), (0, 0)))

    valid2_tb = (t_p[:, None] < counts[None, :]).astype(jnp.float32)
    mask2 = jnp.swapaxes(valid2_tb, 0, 1)[:, :, None]
    h2 = packed
    for pf, pb in rnn1:
        h2 = _bigru_layer(h2, valid2_tb, pf, pb, ones_tb) * mask2

    return _att_norm(h2, att_p, jnp.max(counts))


# speedup vs baseline: 1.4605x; 1.1931x over previous
import functools

import jax
import jax.numpy as jnp
from jax.experimental import pallas as pl
from jax.experimental.pallas import tpu as pltpu


_CHUNK = 128


def _mm_kernel(a_ref, b_ref, o_ref):
    o_ref[...] = jnp.dot(a_ref[...], b_ref[...],
                         preferred_element_type=jnp.float32)


def _mm(a, b, tile_m=1024):
    M, K = a.shape
    N = b.shape[1]
    Mp = ((M + tile_m - 1) // tile_m) * tile_m
    if Mp != M:
        a = jnp.pad(a, ((0, Mp - M), (0, 0)))
    out = pl.pallas_call(
        _mm_kernel,
        grid=(Mp // tile_m,),
        in_specs=[pl.BlockSpec((tile_m, K), lambda i: (i, 0)),
                  pl.BlockSpec((K, N), lambda i: (0, 0))],
        out_specs=pl.BlockSpec((tile_m, N), lambda i: (i, 0)),
        out_shape=jax.ShapeDtypeStruct((Mp, N), jnp.float32),
    )(a, b)
    return out[:M]


def _gru_scan_kernel(x_ref, seg_ref, wih_ref, whh_ref, bih_ref, bhh_ref,
                     o_ref, h_ref, gi_ref, *, reverse):
    @pl.when(pl.program_id(0) == 0)
    def _init():
        h_ref[...] = jnp.zeros_like(h_ref)

    C, B, IN = x_ref.shape
    H = h_ref.shape[1]
    gi = jnp.dot(x_ref[...].reshape(C * B, IN), wih_ref[...],
                 preferred_element_type=jnp.float32) + bih_ref[...]
    gi_ref[...] = gi.reshape(C, B, 3 * H)

    def body(k, h):
        t = (C - 1 - k) if reverse else k
        gh = jnp.dot(h, whh_ref[...],
                     preferred_element_type=jnp.float32) + bhh_ref[...]
        g = gi_ref[pl.ds(t, 1)][0]
        r = jax.nn.sigmoid(g[:, :H] + gh[:, :H])
        z = jax.nn.sigmoid(g[:, H:2 * H] + gh[:, H:2 * H])
        n = jnp.tanh(g[:, 2 * H:] + r * gh[:, 2 * H:])
        hn = (1.0 - z) * n + z * h
        o_ref[pl.ds(t, 1), :, :] = hn[None]
        st = seg_ref[pl.ds(t, 1), :][0]
        return st[:, None] * hn

    h_ref[...] = jax.lax.fori_loop(0, C, body, h_ref[...])


def _gru_scan(x_tbi, seg_tb, p, reverse=False):
    Wih, Whh, bih, bhh = p
    Tp, B, IN = x_tbi.shape
    H = Whh.shape[1]
    G = Tp // _CHUNK
    if reverse:
        tmap = lambda i: (G - 1 - i, 0, 0)
        smap = lambda i: (G - 1 - i, 0)
    else:
        tmap = lambda i: (i, 0, 0)
        smap = lambda i: (i, 0)
    return pl.pallas_call(
        functools.partial(_gru_scan_kernel, reverse=reverse),
        grid=(G,),
        in_specs=[
            pl.BlockSpec((_CHUNK, B, IN), tmap),
            pl.BlockSpec((_CHUNK, B), smap),
            pl.BlockSpec((IN, 3 * H), lambda i: (0, 0)),
            pl.BlockSpec((H, 3 * H), lambda i: (0, 0)),
            pl.BlockSpec((1, 3 * H), lambda i: (0, 0)),
            pl.BlockSpec((1, 3 * H), lambda i: (0, 0)),
        ],
        out_specs=pl.BlockSpec((_CHUNK, B, H), tmap),
        out_shape=jax.ShapeDtypeStruct((Tp, B, H), jnp.float32),
        scratch_shapes=[pltpu.VMEM((B, H), jnp.float32),
                        pltpu.VMEM((_CHUNK, B, 3 * H), jnp.float32)],
    )(x_tbi, seg_tb, Wih.T, Whh.T, bih[None], bhh[None])


def _bigru_kernel(xf_ref, xb_ref, segb_ref,
                  wihf_ref, wihb_ref, whhf_ref, whhb_ref,
                  bif_ref, bib_ref, bhf_ref, bhb_ref,
                  of_ref, ob_ref, hf_ref, hb_ref, gif_ref, gib_ref):
    @pl.when(pl.program_id(0) == 0)
    def _init():
        hf_ref[...] = jnp.zeros_like(hf_ref)
        hb_ref[...] = jnp.zeros_like(hb_ref)

    C, B, IN = xf_ref.shape
    H = hf_ref.shape[1]
    gif_ref[...] = (jnp.dot(xf_ref[...].reshape(C * B, IN), wihf_ref[...],
                            preferred_element_type=jnp.float32)
                    + bif_ref[...]).reshape(C, B, 3 * H)
    gib_ref[...] = (jnp.dot(xb_ref[...].reshape(C * B, IN), wihb_ref[...],
                            preferred_element_type=jnp.float32)
                    + bib_ref[...]).reshape(C, B, 3 * H)

    def body(k, carry):
        hf, hb = carry
        tb = C - 1 - k
        ghf = jnp.dot(hf, whhf_ref[...],
                      preferred_element_type=jnp.float32) + bhf_ref[...]
        ghb = jnp.dot(hb, whhb_ref[...],
                      preferred_element_type=jnp.float32) + bhb_ref[...]
        gf = gif_ref[pl.ds(k, 1)][0]
        gb = gib_ref[pl.ds(tb, 1)][0]
        rf = jax.nn.sigmoid(gf[:, :H] + ghf[:, :H])
        zf = jax.nn.sigmoid(gf[:, H:2 * H] + ghf[:, H:2 * H])
        nf = jnp.tanh(gf[:, 2 * H:] + rf * ghf[:, 2 * H:])
        hnf = (1.0 - zf) * nf + zf * hf
        rb = jax.nn.sigmoid(gb[:, :H] + ghb[:, :H])
        zb = jax.nn.sigmoid(gb[:, H:2 * H] + ghb[:, H:2 * H])
        nb = jnp.tanh(gb[:, 2 * H:] + rb * ghb[:, 2 * H:])
        hnb = (1.0 - zb) * nb + zb * hb
        of_ref[pl.ds(k, 1), :, :] = hnf[None]
        ob_ref[pl.ds(tb, 1), :, :] = hnb[None]
        sb = segb_ref[pl.ds(tb, 1), :][0]
        return hnf, sb[:, None] * hnb

    hf, hb = jax.lax.fori_loop(0, C, body, (hf_ref[...], hb_ref[...]))
    hf_ref[...] = hf
    hb_ref[...] = hb


def _bigru_layer(x_bti, valid_tb, pf, pb, ones_tb):
    x_tbi = jnp.swapaxes(x_bti, 0, 1)
    Wf, Whf, bif, bhf = pf
    Wb, Whb, bib, bhb = pb
    Tp, B, IN = x_tbi.shape
    H = Whf.shape[1]
    chunk = 64 if IN >= 1024 else _CHUNK
    G = Tp // chunk
    fmap3 = lambda i: (i, 0, 0)
    rmap3 = lambda i: (G - 1 - i, 0, 0)
    rmap2 = lambda i: (G - 1 - i, 0)
    zmap = lambda i: (0, 0)
    yf, yb = pl.pallas_call(
        _bigru_kernel,
        grid=(G,),
        in_specs=[
            pl.BlockSpec((chunk, B, IN), fmap3),
            pl.BlockSpec((chunk, B, IN), rmap3),
            pl.BlockSpec((chunk, B), rmap2),
            pl.BlockSpec((IN, 3 * H), zmap),
            pl.BlockSpec((IN, 3 * H), zmap),
            pl.BlockSpec((H, 3 * H), zmap),
            pl.BlockSpec((H, 3 * H), zmap),
            pl.BlockSpec((1, 3 * H), zmap),
            pl.BlockSpec((1, 3 * H), zmap),
            pl.BlockSpec((1, 3 * H), zmap),
            pl.BlockSpec((1, 3 * H), zmap),
        ],
        out_specs=[pl.BlockSpec((chunk, B, H), fmap3),
                   pl.BlockSpec((chunk, B, H), rmap3)],
        out_shape=[jax.ShapeDtypeStruct((Tp, B, H), jnp.float32),
                   jax.ShapeDtypeStruct((Tp, B, H), jnp.float32)],
        scratch_shapes=[pltpu.VMEM((B, H), jnp.float32),
                        pltpu.VMEM((B, H), jnp.float32),
                        pltpu.VMEM((chunk, B, 3 * H), jnp.float32),
                        pltpu.VMEM((chunk, B, 3 * H), jnp.float32)],
    )(x_tbi, x_tbi, valid_tb, Wf.T, Wb.T, Whf.T, Whb.T,
      bif[None], bib[None], bhf[None], bhb[None])
    return jnp.concatenate([jnp.swapaxes(yf, 0, 1),
                            jnp.swapaxes(yb, 0, 1)], axis=-1)


def _vq_kernel(z_ref, cbt_ref, c2_ref, cb_ref, q_ref, idx_ref):
    s = jnp.dot(z_ref[...], cbt_ref[...],
                preferred_element_type=jnp.float32) * (-2.0) + c2_ref[...]
    idx = jnp.argmin(s, axis=1).astype(jnp.int32)
    oh = (jax.lax.broadcasted_iota(jnp.int32, s.shape, 1)
          == idx[:, None]).astype(jnp.float32)
    q_ref[...] = jnp.dot(oh, cb_ref[...], preferred_element_type=jnp.float32)
    idx_ref[...] = idx[:, None]


def _vq(z2d, codebook, tile_m=1024):
    M, D = z2d.shape
    N = codebook.shape[0]
    Mp = ((M + tile_m - 1) // tile_m) * tile_m
    if Mp != M:
        z2d = jnp.pad(z2d, ((0, Mp - M), (0, 0)))
    c2 = jnp.sum(codebook * codebook, axis=1)[None, :]
    q, idx = pl.pallas_call(
        _vq_kernel,
        grid=(Mp // tile_m,),
        in_specs=[pl.BlockSpec((tile_m, D), lambda i: (i, 0)),
                  pl.BlockSpec((D, N), lambda i: (0, 0)),
                  pl.BlockSpec((1, N), lambda i: (0, 0)),
                  pl.BlockSpec((N, D), lambda i: (0, 0))],
        out_specs=[pl.BlockSpec((tile_m, D), lambda i: (i, 0)),
                   pl.BlockSpec((tile_m, 1), lambda i: (i, 0))],
        out_shape=[jax.ShapeDtypeStruct((Mp, D), jnp.float32),
                   jax.ShapeDtypeStruct((Mp, 1), jnp.int32)],
    )(z2d, codebook.T, c2, codebook)
    return q[:M], idx[:M, 0]


def _att_kernel(x_ref, wht_ref, bh_ref, wot_ref, bo_ref, mask_ref, o_ref):
    x = x_ref[0]
    h = jnp.tanh(jnp.dot(x, wht_ref[...],
                         preferred_element_type=jnp.float32) + bh_ref[...])
    a = jnp.dot(h, wot_ref[...],
                preferred_element_type=jnp.float32) + bo_ref[...]
    a = jnp.where(mask_ref[...] > 0, a, -1e30)
    amax = jnp.max(a, axis=0, keepdims=True)
    e = jnp.exp(a - amax)
    alpha = e / jnp.sum(e, axis=0, keepdims=True)
    pooled = jnp.sum(alpha * x, axis=0)
    nrm = jnp.sqrt(jnp.sum(pooled * pooled))
    o_ref[0, 0] = pooled / jnp.maximum(nrm, 1e-12)


def _att_norm(x_btd, att_p, tmax):
    B, Tp, D = x_btd.shape
    Wh, bh, Wo, bo = att_p
    A = Wh.shape[0]
    mask = (jnp.arange(Tp) < tmax).astype(jnp.float32)[:, None]
    return pl.pallas_call(
        _att_kernel,
        grid=(B,),
        in_specs=[
            pl.BlockSpec((1, Tp, D), lambda i: (i, 0, 0)),
            pl.BlockSpec((D, A), lambda i: (0, 0)),
            pl.BlockSpec((1, A), lambda i: (0, 0)),
            pl.BlockSpec((A, D), lambda i: (0, 0)),
            pl.BlockSpec((1, D), lambda i: (0, 0)),
            pl.BlockSpec((Tp, 1), lambda i: (0, 0)),
        ],
        out_specs=pl.BlockSpec((1, 1, D), lambda i: (i, 0, 0)),
        out_shape=jax.ShapeDtypeStruct((B, 1, D), jnp.float32),
    )(x_btd, Wh.T, bh[None], Wo.T, bo[None], mask)[:, 0]


def kernel(input, conv_w, conv_b, rnn0, pack, codebook, rnn1, att_p, l):
    B, Cin, L = input.shape
    O, _, K = conv_w.shape
    stride = 2
    T = (L - K) // stride + 1
    Tp = ((T + _CHUNK - 1) // _CHUNK) * _CHUNK
    l1 = (l - 4) // 2

    idx_t = stride * jnp.arange(T)[:, None] + jnp.arange(K)[None, :]
    win = input[:, :, idx_t]
    win = jnp.transpose(win, (0, 2, 3, 1)).reshape(B * T, K * Cin)
    wmat = jnp.transpose(conv_w, (2, 1, 0)).reshape(K * Cin, O)
    x = (_mm(win, wmat) + conv_b[None]).reshape(B, T, O)
    x = jnp.pad(x, ((0, 0), (0, Tp - T), (0, 0)))

    t_p = jnp.arange(Tp)
    valid1_tb = (t_p[:, None] < l1[None, :]).astype(jnp.float32)
    mask1 = jnp.swapaxes(valid1_tb, 0, 1)[:, :, None]
    ones_tb = jnp.ones((Tp, B), jnp.float32)

    h = x
    for pf, pb in rnn0:
        h = _bigru_layer(h, valid1_tb, pf, pb, ones_tb) * mask1

    D = codebook.shape[1]
    q2d, idx_flat = _vq(h.reshape(B * Tp, D), codebook)
    zq = q2d.reshape(B, Tp, D)
    idx = idx_flat.reshape(B, Tp)[:, :T]

    roll = jnp.roll(idx, 1, axis=1).at[:, 0].set(-1)
    seg = jnp.roll((idx == roll).astype(jnp.float32), -1, axis=1)
    Tmax1 = jnp.max(l1)
    tt = jnp.arange(T)
    seg = jnp.where(tt[None, :] == Tmax1 - 1, 0.0, seg)
    seg_p = jnp.pad(seg, ((0, 0), (0, Tp - T)))

    hs = _gru_scan(jnp.swapaxes(zq, 0, 1), jnp.swapaxes(seg_p, 0, 1), pack)
    hs = jnp.swapaxes(hs, 0, 1)[:, :T]

    m = (seg == 0) & (tt[None, :] < l1[:, None])
    counts = m.sum(1).astype(jnp.int32)
    src = jnp.argsort(jnp.where(m, tt[None, :], T), axis=1)
    packed = jnp.take_along_axis(hs, src[:, :, None], axis=1)
    packed = packed * (tt[None, :] < counts[:, None])[:, :, None]
    packed = jnp.pad(packed, ((0, 0), (0, Tp - T), (0, 0)))

    valid2_tb = (t_p[:, None] < counts[None, :]).astype(jnp.float32)
    mask2 = jnp.swapaxes(valid2_tb, 0, 1)[:, :, None]
    h2 = packed
    for pf, pb in rnn1:
        h2 = _bigru_layer(h2, valid2_tb, pf, pb, ones_tb) * mask2

    return _att_norm(h2, att_p, jnp.max(counts))
